# baseline (speedup 1.0000x reference)
import functools

import jax
import jax.numpy as jnp
from jax.experimental import pallas as pl
from jax.experimental.pallas import tpu as pltpu

_E = 8
_K = 2


def _router_kernel(n_tokens, x_ref, w_ref, rw_ref, se_ref, aux_ref,
                   cnt_ref, ps_ref):
    i = pl.program_id(0)
    n_i = pl.num_programs(0)
    x = x_ref[...]
    w = w_ref[...]
    logits = jax.lax.dot_general(
        w, x, (((1,), (1,)), ((), ())),
        preferred_element_type=jnp.float32)

    eidx = jax.lax.broadcasted_iota(jnp.int32, logits.shape, 0)
    m1 = jnp.max(logits, axis=0, keepdims=True)
    i1 = jnp.min(jnp.where(logits == m1, eidx, _E), axis=0, keepdims=True)
    masked = jnp.where(eidx == i1, -jnp.inf, logits)
    m2 = jnp.max(masked, axis=0, keepdims=True)
    i2 = jnp.min(jnp.where(masked == m2, eidx, _E), axis=0, keepdims=True)

    e2 = jnp.exp(m2 - m1)
    denom = 1.0 + e2
    rw_ref[...] = jnp.concatenate([1.0 / denom, e2 / denom], axis=0)
    se_ref[...] = jnp.concatenate([i1, i2], axis=0)

    ex = jnp.exp(logits - m1)
    probs = ex / jnp.sum(ex, axis=0, keepdims=True)
    hit = ((eidx == i1) | (eidx == i2)).astype(jnp.float32)

    @pl.when(i == 0)
    def _init():
        cnt_ref[...] = jnp.zeros_like(cnt_ref)
        ps_ref[...] = jnp.zeros_like(ps_ref)

    cnt_ref[...] += jnp.sum(hit, axis=1, keepdims=True)
    ps_ref[...] += jnp.sum(probs, axis=1, keepdims=True)

    @pl.when(i == n_i - 1)
    def _finish():
        f = cnt_ref[...] / (n_tokens * _K)
        p_mean = ps_ref[...] / n_tokens
        aux_ref[...] = jnp.reshape(_E * jnp.sum(f * p_mean), (1, 1))


def kernel(hidden_states, W):
    n, d = hidden_states.shape
    block = 8192
    grid = (n // block,)

    rw, se, aux = pl.pallas_call(
        functools.partial(_router_kernel, n),
        grid=grid,
        in_specs=[
            pl.BlockSpec((block, d), lambda i: (i, 0)),
            pl.BlockSpec((_E, d), lambda i: (0, 0)),
        ],
        out_specs=[
            pl.BlockSpec((_K, block), lambda i: (0, i)),
            pl.BlockSpec((_K, block), lambda i: (0, i)),
            pl.BlockSpec((1, 1), lambda i: (0, 0)),
        ],
        out_shape=[
            jax.ShapeDtypeStruct((_K, n), jnp.float32),
            jax.ShapeDtypeStruct((_K, n), jnp.int32),
            jax.ShapeDtypeStruct((1, 1), jnp.float32),
        ],
        scratch_shapes=[
            pltpu.VMEM((_E, 1), jnp.float32),
            pltpu.VMEM((_E, 1), jnp.float32),
        ],
        compiler_params=pltpu.CompilerParams(
            dimension_semantics=("arbitrary",),
        ),
    )(hidden_states, W)
    return (rw.T, se.T, aux.reshape(()))


# speedup vs baseline: 1.0875x; 1.0875x over previous
import functools

import jax
import jax.numpy as jnp
from jax.experimental import pallas as pl
from jax.experimental.pallas import tpu as pltpu

_E = 8
_K = 2


def _router_kernel(n_tokens, x_ref, w_ref, rw_ref, se_ref, aux_ref,
                   cnt_ref, ps_ref):
    i = pl.program_id(0)
    n_i = pl.num_programs(0)
    x = x_ref[...]
    w = w_ref[...]
    logits = jax.lax.dot_general(
        w, x, (((1,), (1,)), ((), ())),
        preferred_element_type=jnp.float32)

    eidx = jax.lax.broadcasted_iota(jnp.int32, logits.shape, 0)
    m1 = jnp.max(logits, axis=0, keepdims=True)
    i1 = jnp.min(jnp.where(logits == m1, eidx, _E), axis=0, keepdims=True)
    masked = jnp.where(eidx == i1, -jnp.inf, logits)
    m2 = jnp.max(masked, axis=0, keepdims=True)
    i2 = jnp.min(jnp.where(masked == m2, eidx, _E), axis=0, keepdims=True)

    e2 = jnp.exp(m2 - m1)
    denom = 1.0 + e2
    rw_ref[...] = jnp.concatenate([1.0 / denom, e2 / denom], axis=0)
    se_ref[...] = jnp.concatenate([i1, i2], axis=0)

    ex = jnp.exp(logits - m1)
    probs = ex / jnp.sum(ex, axis=0, keepdims=True)
    hit = ((eidx == i1) | (eidx == i2)).astype(jnp.float32)

    @pl.when(i == 0)
    def _init():
        cnt_ref[...] = jnp.zeros_like(cnt_ref)
        ps_ref[...] = jnp.zeros_like(ps_ref)

    cnt_ref[...] += jnp.sum(hit, axis=1, keepdims=True)
    ps_ref[...] += jnp.sum(probs, axis=1, keepdims=True)

    @pl.when(i == n_i - 1)
    def _finish():
        f = cnt_ref[...] / (n_tokens * _K)
        p_mean = ps_ref[...] / n_tokens
        aux_ref[...] = jnp.reshape(_E * jnp.sum(f * p_mean), (1, 1))


def kernel(hidden_states, W):
    n, d = hidden_states.shape
    block = 4096
    grid = (n // block,)

    rw, se, aux = pl.pallas_call(
        functools.partial(_router_kernel, n),
        grid=grid,
        in_specs=[
            pl.BlockSpec((block, d), lambda i: (i, 0)),
            pl.BlockSpec((_E, d), lambda i: (0, 0)),
        ],
        out_specs=[
            pl.BlockSpec((_K, block), lambda i: (0, i)),
            pl.BlockSpec((_K, block), lambda i: (0, i)),
            pl.BlockSpec((1, 1), lambda i: (0, 0)),
        ],
        out_shape=[
            jax.ShapeDtypeStruct((_K, n), jnp.float32),
            jax.ShapeDtypeStruct((_K, n), jnp.int32),
            jax.ShapeDtypeStruct((1, 1), jnp.float32),
        ],
        scratch_shapes=[
            pltpu.VMEM((_E, 1), jnp.float32),
            pltpu.VMEM((_E, 1), jnp.float32),
        ],
        compiler_params=pltpu.CompilerParams(
            dimension_semantics=("arbitrary",),
        ),
    )(hidden_states, W)
    return (rw.T, se.T, aux.reshape(()))
